# baseline (speedup 1.0000x reference)
import functools

import jax
import jax.numpy as jnp
from jax import lax
from jax.experimental import pallas as pl
from jax.experimental.pallas import tpu as pltpu
from jax.experimental.pallas import tpu_sc as plsc

N = 10000
NP = 10240
E = 320000
EPAD = 327680
CH = EPAD // 128
NCH_T = CH // 16
NCH_D = CH // 32
RPT = NP // 16
BN = 512

_MESH = dict(core_axis_name="c", subcore_axis_name="s", num_cores=2,
             num_subcores=16)


def _deg_body(dstr, ones_h, zeros_h, out, dst_v, ones_v, acc, sem):
    c = lax.axis_index("c")
    s = lax.axis_index("s")
    wid = c * 16 + s
    pltpu.sync_copy(dstr.at[pl.ds(wid * NCH_D, NCH_D)], dst_v)
    pltpu.sync_copy(ones_h, ones_v)
    pltpu.sync_copy(zeros_h, acc.at[pl.ds(s * RPT, RPT)])
    plsc.subcore_barrier()

    def chunk(t, carry):
        pltpu.sync_copy(ones_v, acc.at[dst_v.at[t]], add=True)
        return carry

    lax.fori_loop(0, NCH_D, chunk, 0)
    plsc.subcore_barrier()
    pltpu.sync_copy(acc.at[pl.ds(s * RPT, RPT)],
                    out.at[pl.ds(c * NP + s * RPT, RPT)])


_deg_kernel = functools.partial(
    pl.kernel,
    out_type=jax.ShapeDtypeStruct((2 * NP,), jnp.float32),
    mesh=plsc.VectorSubcoreMesh(**_MESH),
    scratch_types=[
        pltpu.VMEM((NCH_D, 128), jnp.int32),
        pltpu.VMEM((128,), jnp.float32),
        pltpu.VMEM_SHARED((NP,), jnp.float32),
        pltpu.SemaphoreType.DMA,
    ],
)(_deg_body)


CHUNK = 64
CH64 = EPAD // CHUNK
KB = 32
NBUF = 4


def _agg_body(edge_split, table, src2r, dstr, zeros_h, out,
              src_v, dst_v, acc, *bufs_and_sems):
    bufs = bufs_and_sems[:NBUF]
    sems = bufs_and_sems[NBUF:]
    c = lax.axis_index("c")
    s = lax.axis_index("s")
    if edge_split:
        src_base = (c * 16 + s) * (CH64 // 32)
        dst_base = src_base
        nb = CH64 // 32 // KB
    else:
        src_base = c * CH64 + s * (CH64 // 16)
        dst_base = s * (CH64 // 16)
        nb = CH64 // 16 // KB
    pltpu.sync_copy(zeros_h, acc.at[pl.ds(s * RPT, RPT)])
    plsc.subcore_barrier()

    def outer(b, carry):
        pltpu.sync_copy(src2r.at[pl.ds(src_base + b * KB, KB)], src_v)
        pltpu.sync_copy(dstr.at[pl.ds(dst_base + b * KB, KB)], dst_v)
        for k in range(NBUF):
            pltpu.async_copy(table.at[src_v.at[k]], bufs[k], sems[k])

        def grp(t, carry2):
            j = t * NBUF
            for k in range(NBUF):
                pltpu.make_async_copy(table.at[src_v.at[j + k]], bufs[k],
                                      sems[k]).wait()
                pltpu.sync_copy(bufs[k], acc.at[dst_v.at[j + k]], add=True)

                @pl.when(j + k + NBUF < KB)
                def _():
                    pltpu.async_copy(table.at[src_v.at[j + k + NBUF]],
                                     bufs[k], sems[k])

            return carry2

        lax.fori_loop(0, KB // NBUF, grp, 0)
        return carry

    lax.fori_loop(0, nb, outer, 0)
    plsc.subcore_barrier()
    pltpu.sync_copy(acc.at[pl.ds(s * RPT, RPT)],
                    out.at[pl.ds(c * NP + s * RPT, RPT)])


def _make_agg_kernel(edge_split):
    return functools.partial(
        pl.kernel,
        out_type=jax.ShapeDtypeStruct((2 * NP, 128), jnp.float32),
        mesh=plsc.VectorSubcoreMesh(**_MESH),
        scratch_types=[
            pltpu.VMEM((KB, CHUNK), jnp.int32),
            pltpu.VMEM((KB, CHUNK), jnp.int32),
            pltpu.VMEM_SHARED((NP, 128), jnp.float32),
        ] + [pltpu.VMEM((CHUNK, 128), jnp.float32)] * NBUF
          + [pltpu.SemaphoreType.DMA] * NBUF,
    )(functools.partial(_agg_body, edge_split))


_agg_feat = _make_agg_kernel(False)
_agg_edge = _make_agg_kernel(True)


def _dinv_block(degp_blk, i):
    dsum = degp_blk[:, 0:1] + degp_blk[:, 1:2] + 1.0
    rowid = lax.broadcasted_iota(jnp.int32, (BN, 1), 0) + i * BN
    return jnp.where(rowid < N, lax.rsqrt(dsum), 0.0)


def _tc1_body(x_ref, degp_ref, w0_ref, w1_ref, y2_ref, x0_ref):
    i = pl.program_id(0)
    xw = jnp.dot(x_ref[...], w0_ref[...], preferred_element_type=jnp.float32)
    dinv = _dinv_block(degp_ref[...], i)
    y = xw * dinv
    y2_ref[0] = y[:, :128]
    y2_ref[1] = y[:, 128:]
    x0_ref[...] = jnp.dot(xw, w1_ref[...], preferred_element_type=jnp.float32)


def _tc2_body(acc_ref, y2_ref, degp_ref, w1_ref, b0_ref, h1_ref, z_ref):
    i = pl.program_id(0)
    accf = jnp.concatenate([acc_ref[0], acc_ref[1]], axis=1)
    yf = jnp.concatenate([y2_ref[0], y2_ref[1]], axis=1)
    dinv = _dinv_block(degp_ref[...], i)
    h = jnp.maximum(dinv * (accf + yf) + b0_ref[...], 0.0)
    hw1 = jnp.dot(h, w1_ref[...], preferred_element_type=jnp.float32)
    h1_ref[...] = jnp.maximum(hw1, 0.0)
    z_ref[...] = hw1 * dinv


def _tc3_body(acc_ref, z_ref, degp_ref, b1_ref, out_ref):
    i = pl.program_id(0)
    accf = acc_ref[0] + acc_ref[1] + z_ref[...]
    dinv = _dinv_block(degp_ref[...], i)
    out_ref[...] = jnp.maximum(dinv * accf + b1_ref[...], 0.0)


_GRID = NP // BN


def _bs(shape, index_map):
    return pl.BlockSpec(shape, index_map)


_tc1 = pl.pallas_call(
    _tc1_body,
    grid=(_GRID,),
    in_specs=[
        _bs((BN, 128), lambda i: (i, 0)),
        _bs((BN, 2), lambda i: (i, 0)),
        _bs((128, 256), lambda i: (0, 0)),
        _bs((256, 128), lambda i: (0, 0)),
    ],
    out_specs=[
        _bs((2, BN, 128), lambda i: (0, i, 0)),
        _bs((BN, 128), lambda i: (i, 0)),
    ],
    out_shape=[
        jax.ShapeDtypeStruct((2, NP, 128), jnp.float32),
        jax.ShapeDtypeStruct((N, 128), jnp.float32),
    ],
)

_tc2 = pl.pallas_call(
    _tc2_body,
    grid=(_GRID,),
    in_specs=[
        _bs((2, BN, 128), lambda i: (0, i, 0)),
        _bs((2, BN, 128), lambda i: (0, i, 0)),
        _bs((BN, 2), lambda i: (i, 0)),
        _bs((256, 128), lambda i: (0, 0)),
        _bs((1, 256), lambda i: (0, 0)),
    ],
    out_specs=[
        _bs((BN, 128), lambda i: (i, 0)),
        _bs((BN, 128), lambda i: (i, 0)),
    ],
    out_shape=[
        jax.ShapeDtypeStruct((N, 128), jnp.float32),
        jax.ShapeDtypeStruct((NP, 128), jnp.float32),
    ],
)

_tc3 = pl.pallas_call(
    _tc3_body,
    grid=(_GRID,),
    in_specs=[
        _bs((2, BN, 128), lambda i: (0, i, 0)),
        _bs((BN, 128), lambda i: (i, 0)),
        _bs((BN, 2), lambda i: (i, 0)),
        _bs((1, 128), lambda i: (0, 0)),
    ],
    out_specs=_bs((BN, 128), lambda i: (i, 0)),
    out_shape=jax.ShapeDtypeStruct((N, 128), jnp.float32),
)


def kernel(x, edge_index, W0, b0, W1, b1):
    src = edge_index[0]
    dst = edge_index[1]

    pad = N + (jnp.arange(EPAD - E, dtype=jnp.int32) % (NP - N))
    srcp = jnp.concatenate([src, pad])
    dstp = jnp.concatenate([dst, pad])
    src2r = jnp.concatenate([srcp, srcp + NP]).reshape(2 * CH64, CHUNK)
    srcr = srcp.reshape(CH64, CHUNK)
    dstr = dstp.reshape(CH64, CHUNK)
    dstr128 = dstp.reshape(CH, 128)

    x_pad = jnp.concatenate(
        [x, jnp.zeros((NP - N, x.shape[1]), x.dtype)], axis=0)

    zeros_r = jnp.zeros((RPT,), jnp.float32)
    zeros_r128 = jnp.zeros((RPT, 128), jnp.float32)
    ones128 = jnp.ones((128,), jnp.float32)

    degf = _deg_kernel(dstr128, ones128, zeros_r)
    degp = jnp.stack([degf[:NP], degf[NP:]], axis=1)

    y2, x0 = _tc1(x_pad, degp, W0, W1)
    acc1 = _agg_feat(y2.reshape(2 * NP, 128), src2r, dstr, zeros_r128)
    h1, z = _tc2(acc1.reshape(2, NP, 128), y2, degp, W1,
                 b0.reshape(1, 256))
    acc2 = _agg_edge(z, srcr, dstr, zeros_r128)
    out = _tc3(acc2.reshape(2, NP, 128), z, degp, b1.reshape(1, 128))
    return (out, x0, h1)


# speedup vs baseline: 1.0075x; 1.0075x over previous
import functools

import jax
import jax.numpy as jnp
from jax import lax
from jax.experimental import pallas as pl
from jax.experimental.pallas import tpu as pltpu
from jax.experimental.pallas import tpu_sc as plsc

N = 10000
NP = 10240
E = 320000
EPAD = 327680
CH = EPAD // 128
NCH_T = CH // 16
NCH_D = CH // 32
RPT = NP // 16
BN = 512

_MESH = dict(core_axis_name="c", subcore_axis_name="s", num_cores=2,
             num_subcores=16)


def _deg_body(dstr, ones_h, zeros_h, out, dst_v, ones_v, acc, sem):
    c = lax.axis_index("c")
    s = lax.axis_index("s")
    wid = c * 16 + s
    pltpu.sync_copy(dstr.at[pl.ds(wid * NCH_D, NCH_D)], dst_v)
    pltpu.sync_copy(ones_h, ones_v)
    pltpu.sync_copy(zeros_h, acc.at[pl.ds(s * RPT, RPT)])
    plsc.subcore_barrier()

    def chunk(t, carry):
        pltpu.sync_copy(ones_v, acc.at[dst_v.at[t]], add=True)
        return carry

    lax.fori_loop(0, NCH_D, chunk, 0)
    plsc.subcore_barrier()
    pltpu.sync_copy(acc.at[pl.ds(s * RPT, RPT)],
                    out.at[pl.ds(c * NP + s * RPT, RPT)])


_deg_kernel = functools.partial(
    pl.kernel,
    out_type=jax.ShapeDtypeStruct((2 * NP,), jnp.float32),
    mesh=plsc.VectorSubcoreMesh(**_MESH),
    scratch_types=[
        pltpu.VMEM((NCH_D, 128), jnp.int32),
        pltpu.VMEM((128,), jnp.float32),
        pltpu.VMEM_SHARED((NP,), jnp.float32),
        pltpu.SemaphoreType.DMA,
    ],
)(_deg_body)


CHUNK = 64
CH64 = EPAD // CHUNK
KB = 32
NBUF = 4


def _agg_body(edge_split, table, src2r, dstr, zeros_h, out,
              src_v, dst_v, acc, *bufs_and_sems):
    bufs = bufs_and_sems[:NBUF]
    sems = bufs_and_sems[NBUF:]
    c = lax.axis_index("c")
    s = lax.axis_index("s")
    if edge_split:
        src_base = (c * 16 + s) * (CH64 // 32)
        dst_base = src_base
        nb = CH64 // 32 // KB
    else:
        src_base = c * CH64 + s * (CH64 // 16)
        dst_base = s * (CH64 // 16)
        nb = CH64 // 16 // KB
    pltpu.sync_copy(zeros_h, acc.at[pl.ds(s * RPT, RPT)])
    plsc.subcore_barrier()

    def outer(b, carry):
        pltpu.sync_copy(src2r.at[pl.ds(src_base + b * KB, KB)], src_v)
        pltpu.sync_copy(dstr.at[pl.ds(dst_base + b * KB, KB)], dst_v)
        for k in range(NBUF):
            pltpu.async_copy(table.at[src_v.at[k]], bufs[k], sems[k])

        def grp(t, carry2):
            j = t * NBUF
            for k in range(NBUF):
                pltpu.make_async_copy(table.at[src_v.at[j + k]], bufs[k],
                                      sems[k]).wait()
                pltpu.sync_copy(bufs[k], acc.at[dst_v.at[j + k]], add=True)

                @pl.when(j + k + NBUF < KB)
                def _():
                    pltpu.async_copy(table.at[src_v.at[j + k + NBUF]],
                                     bufs[k], sems[k])

            return carry2

        lax.fori_loop(0, KB // NBUF, grp, 0)
        return carry

    lax.fori_loop(0, nb, outer, 0)
    plsc.subcore_barrier()
    pltpu.sync_copy(acc.at[pl.ds(s * RPT, RPT)],
                    out.at[pl.ds(c * NP + s * RPT, RPT)])


def _make_agg_kernel(edge_split):
    return functools.partial(
        pl.kernel,
        out_type=jax.ShapeDtypeStruct((2 * NP, 128), jnp.float32),
        mesh=plsc.VectorSubcoreMesh(**_MESH),
        scratch_types=[
            pltpu.VMEM((KB, CHUNK), jnp.int32),
            pltpu.VMEM((KB, CHUNK), jnp.int32),
            pltpu.VMEM_SHARED((NP, 128), jnp.float32),
        ] + [pltpu.VMEM((CHUNK, 128), jnp.float32)] * NBUF
          + [pltpu.SemaphoreType.DMA] * NBUF,
    )(functools.partial(_agg_body, edge_split))


_agg_feat = _make_agg_kernel(False)
_agg_edge = _make_agg_kernel(True)


def _dinv_block(degp_blk, i):
    dsum = degp_blk[:, 0:1] + degp_blk[:, 1:2] + 1.0
    rowid = lax.broadcasted_iota(jnp.int32, (BN, 1), 0) + i * BN
    return jnp.where(rowid < N, lax.rsqrt(dsum), 0.0)


def _tc1_body(x_ref, degp_ref, w0_ref, y2_ref):
    i = pl.program_id(0)
    xw = jnp.dot(x_ref[...], w0_ref[...], preferred_element_type=jnp.float32)
    dinv = _dinv_block(degp_ref[...], i)
    y = xw * dinv
    y2_ref[0] = y[:, :128]
    y2_ref[1] = y[:, 128:]


def _tc0_body(x_ref, w0_ref, w1_ref, x0_ref):
    xw = jnp.dot(x_ref[...], w0_ref[...], preferred_element_type=jnp.float32)
    x0_ref[...] = jnp.dot(xw, w1_ref[...], preferred_element_type=jnp.float32)


def _tcr_body(hw1_ref, h1_ref):
    h1_ref[...] = jnp.maximum(hw1_ref[...], 0.0)


def _tc2_body(acc_ref, y2_ref, degp_ref, w1_ref, b0_ref, hw1_ref, z_ref):
    i = pl.program_id(0)
    accf = jnp.concatenate([acc_ref[0], acc_ref[1]], axis=1)
    yf = jnp.concatenate([y2_ref[0], y2_ref[1]], axis=1)
    dinv = _dinv_block(degp_ref[...], i)
    h = jnp.maximum(dinv * (accf + yf) + b0_ref[...], 0.0)
    hw1 = jnp.dot(h, w1_ref[...], preferred_element_type=jnp.float32)
    hw1_ref[...] = hw1
    z_ref[...] = hw1 * dinv


def _tc3_body(acc_ref, z_ref, degp_ref, b1_ref, out_ref):
    i = pl.program_id(0)
    accf = acc_ref[0] + acc_ref[1] + z_ref[...]
    dinv = _dinv_block(degp_ref[...], i)
    out_ref[...] = jnp.maximum(dinv * accf + b1_ref[...], 0.0)


_GRID = NP // BN


def _bs(shape, index_map):
    return pl.BlockSpec(shape, index_map)


_tc1 = pl.pallas_call(
    _tc1_body,
    grid=(_GRID,),
    in_specs=[
        _bs((BN, 128), lambda i: (i, 0)),
        _bs((BN, 2), lambda i: (i, 0)),
        _bs((128, 256), lambda i: (0, 0)),
    ],
    out_specs=_bs((2, BN, 128), lambda i: (0, i, 0)),
    out_shape=jax.ShapeDtypeStruct((2, NP, 128), jnp.float32),
)

_tc0 = pl.pallas_call(
    _tc0_body,
    grid=(_GRID,),
    in_specs=[
        _bs((BN, 128), lambda i: (i, 0)),
        _bs((128, 256), lambda i: (0, 0)),
        _bs((256, 128), lambda i: (0, 0)),
    ],
    out_specs=_bs((BN, 128), lambda i: (i, 0)),
    out_shape=jax.ShapeDtypeStruct((N, 128), jnp.float32),
)

_tcr = pl.pallas_call(
    _tcr_body,
    grid=(_GRID,),
    in_specs=[_bs((BN, 128), lambda i: (i, 0))],
    out_specs=_bs((BN, 128), lambda i: (i, 0)),
    out_shape=jax.ShapeDtypeStruct((N, 128), jnp.float32),
)

_tc2 = pl.pallas_call(
    _tc2_body,
    grid=(_GRID,),
    in_specs=[
        _bs((2, BN, 128), lambda i: (0, i, 0)),
        _bs((2, BN, 128), lambda i: (0, i, 0)),
        _bs((BN, 2), lambda i: (i, 0)),
        _bs((256, 128), lambda i: (0, 0)),
        _bs((1, 256), lambda i: (0, 0)),
    ],
    out_specs=[
        _bs((BN, 128), lambda i: (i, 0)),
        _bs((BN, 128), lambda i: (i, 0)),
    ],
    out_shape=[
        jax.ShapeDtypeStruct((NP, 128), jnp.float32),
        jax.ShapeDtypeStruct((NP, 128), jnp.float32),
    ],
)

_tc3 = pl.pallas_call(
    _tc3_body,
    grid=(_GRID,),
    in_specs=[
        _bs((2, BN, 128), lambda i: (0, i, 0)),
        _bs((BN, 128), lambda i: (i, 0)),
        _bs((BN, 2), lambda i: (i, 0)),
        _bs((1, 128), lambda i: (0, 0)),
    ],
    out_specs=_bs((BN, 128), lambda i: (i, 0)),
    out_shape=jax.ShapeDtypeStruct((N, 128), jnp.float32),
)


def kernel(x, edge_index, W0, b0, W1, b1):
    src = edge_index[0]
    dst = edge_index[1]

    pad = N + (jnp.arange(EPAD - E, dtype=jnp.int32) % (NP - N))
    srcp = jnp.concatenate([src, pad])
    dstp = jnp.concatenate([dst, pad])
    src2r = jnp.concatenate([srcp, srcp + NP]).reshape(2 * CH64, CHUNK)
    srcr = srcp.reshape(CH64, CHUNK)
    dstr = dstp.reshape(CH64, CHUNK)
    dstr128 = dstp.reshape(CH, 128)

    x_pad = jnp.concatenate(
        [x, jnp.zeros((NP - N, x.shape[1]), x.dtype)], axis=0)

    zeros_r = jnp.zeros((RPT,), jnp.float32)
    zeros_r128 = jnp.zeros((RPT, 128), jnp.float32)
    ones128 = jnp.ones((128,), jnp.float32)

    degf = _deg_kernel(dstr128, ones128, zeros_r)
    degp = jnp.stack([degf[:NP], degf[NP:]], axis=1)

    y2 = _tc1(x_pad, degp, W0)
    acc1 = _agg_feat(y2.reshape(2 * NP, 128), src2r, dstr, zeros_r128)
    x0 = _tc0(x_pad, W0, W1)
    hw1, z = _tc2(acc1.reshape(2, NP, 128), y2, degp, W1,
                  b0.reshape(1, 256))
    acc2 = _agg_edge(z, srcr, dstr, zeros_r128)
    h1 = _tcr(hw1)
    out = _tc3(acc2.reshape(2, NP, 128), z, degp, b1.reshape(1, 128))
    return (out, x0, h1)


# speedup vs baseline: 1.0312x; 1.0236x over previous
import functools

import jax
import jax.numpy as jnp
from jax import lax
from jax.experimental import pallas as pl
from jax.experimental.pallas import tpu as pltpu
from jax.experimental.pallas import tpu_sc as plsc

N = 10000
NP = 10240
E = 320000
EPAD = 327680
CH = EPAD // 128
NCH_T = CH // 16
NCH_D = CH // 32
RPT = NP // 16
BN = 512

_MESH = dict(core_axis_name="c", subcore_axis_name="s", num_cores=2,
             num_subcores=16)


def _deg_body(dstr, ones_h, zeros_h, out, dst_v, ones_v, acc, sem):
    c = lax.axis_index("c")
    s = lax.axis_index("s")
    wid = c * 16 + s
    pltpu.sync_copy(dstr.at[pl.ds(wid * NCH_D, NCH_D)], dst_v)
    pltpu.sync_copy(ones_h, ones_v)
    pltpu.sync_copy(zeros_h, acc.at[pl.ds(s * RPT, RPT)])
    plsc.subcore_barrier()

    def chunk(t, carry):
        pltpu.sync_copy(ones_v, acc.at[dst_v.at[t]], add=True)
        return carry

    lax.fori_loop(0, NCH_D, chunk, 0)
    plsc.subcore_barrier()
    pltpu.sync_copy(acc.at[pl.ds(s * RPT, RPT)],
                    out.at[pl.ds(c * NP + s * RPT, RPT)])


_deg_kernel = functools.partial(
    pl.kernel,
    out_type=jax.ShapeDtypeStruct((2 * NP,), jnp.float32),
    mesh=plsc.VectorSubcoreMesh(**_MESH),
    scratch_types=[
        pltpu.VMEM((NCH_D, 128), jnp.int32),
        pltpu.VMEM((128,), jnp.float32),
        pltpu.VMEM_SHARED((NP,), jnp.float32),
        pltpu.SemaphoreType.DMA,
    ],
)(_deg_body)


CHUNK = 64
CH64 = EPAD // CHUNK
KB = 32
NBUF = 4


def _agg_body(edge_split, kb, table, src2r, dstr, zeros_h, out,
              src_v0, dst_v0, src_v1, dst_v1, acc, si, *bufs_and_sems):
    bufs = bufs_and_sems[:NBUF]
    sems = bufs_and_sems[NBUF:]
    c = lax.axis_index("c")
    s = lax.axis_index("s")
    if edge_split:
        src_base = (c * 16 + s) * (CH64 // 32)
        dst_base = src_base
        nb = CH64 // 32 // kb
    else:
        src_base = c * CH64 + s * (CH64 // 16)
        dst_base = s * (CH64 // 16)
        nb = CH64 // 16 // kb
    pltpu.sync_copy(zeros_h, acc.at[pl.ds(s * RPT, RPT)])
    pltpu.sync_copy(src2r.at[pl.ds(src_base, kb)], src_v0)
    pltpu.sync_copy(dstr.at[pl.ds(dst_base, kb)], dst_v0)
    plsc.subcore_barrier()

    def half(b, src_v, dst_v, nsv, ndv):
        nsrc = src2r.at[pl.ds(src_base + (b + 1) * kb, kb)]
        ndst = dstr.at[pl.ds(dst_base + (b + 1) * kb, kb)]

        @pl.when(b + 1 < nb)
        def _():
            pltpu.async_copy(nsrc, nsv, si)
            pltpu.async_copy(ndst, ndv, si)

        for k in range(NBUF):
            pltpu.async_copy(table.at[src_v.at[k]], bufs[k], sems[k])

        def grp(t, carry2):
            j = t * NBUF
            for k in range(NBUF):
                pltpu.make_async_copy(table.at[src_v.at[j + k]], bufs[k],
                                      sems[k]).wait()
                pltpu.sync_copy(bufs[k], acc.at[dst_v.at[j + k]], add=True)

                @pl.when(j + k + NBUF < kb)
                def _():
                    pltpu.async_copy(table.at[src_v.at[j + k + NBUF]],
                                     bufs[k], sems[k])

            return carry2

        lax.fori_loop(0, kb // NBUF, grp, 0)

        @pl.when(b + 1 < nb)
        def _():
            pltpu.make_async_copy(nsrc, nsv, si).wait()
            pltpu.make_async_copy(ndst, ndv, si).wait()

    def outer(b2, carry):
        half(2 * b2, src_v0, dst_v0, src_v1, dst_v1)
        half(2 * b2 + 1, src_v1, dst_v1, src_v0, dst_v0)
        return carry

    lax.fori_loop(0, nb // 2, outer, 0)
    plsc.subcore_barrier()
    pltpu.sync_copy(acc.at[pl.ds(s * RPT, RPT)],
                    out.at[pl.ds(c * NP + s * RPT, RPT)])


def _make_agg_kernel(edge_split):
    kb = KB // 2 if edge_split else KB
    return functools.partial(
        pl.kernel,
        out_type=jax.ShapeDtypeStruct((2 * NP, 128), jnp.float32),
        mesh=plsc.VectorSubcoreMesh(**_MESH),
        scratch_types=[
            pltpu.VMEM((kb, CHUNK), jnp.int32),
            pltpu.VMEM((kb, CHUNK), jnp.int32),
            pltpu.VMEM((kb, CHUNK), jnp.int32),
            pltpu.VMEM((kb, CHUNK), jnp.int32),
            pltpu.VMEM_SHARED((NP, 128), jnp.float32),
            pltpu.SemaphoreType.DMA,
        ] + [pltpu.VMEM((CHUNK, 128), jnp.float32)] * NBUF
          + [pltpu.SemaphoreType.DMA] * NBUF,
    )(functools.partial(_agg_body, edge_split, kb))


_agg_feat = _make_agg_kernel(False)
_agg_edge = _make_agg_kernel(True)


def _dinv_block(degp_blk, i):
    dsum = degp_blk[:, 0:1] + degp_blk[:, 1:2] + 1.0
    rowid = lax.broadcasted_iota(jnp.int32, (BN, 1), 0) + i * BN
    return jnp.where(rowid < N, lax.rsqrt(dsum), 0.0)


def _tc1_body(x_ref, degp_ref, w0_ref, y2_ref):
    i = pl.program_id(0)
    xw = jnp.dot(x_ref[...], w0_ref[...], preferred_element_type=jnp.float32)
    dinv = _dinv_block(degp_ref[...], i)
    y = xw * dinv
    y2_ref[0] = y[:, :128]
    y2_ref[1] = y[:, 128:]


def _tc0_body(x_ref, w0_ref, w1_ref, x0_ref):
    xw = jnp.dot(x_ref[...], w0_ref[...], preferred_element_type=jnp.float32)
    x0_ref[...] = jnp.dot(xw, w1_ref[...], preferred_element_type=jnp.float32)


def _tcr_body(hw1_ref, h1_ref):
    h1_ref[...] = jnp.maximum(hw1_ref[...], 0.0)


def _tc2_body(acc_ref, y2_ref, degp_ref, w1_ref, b0_ref, hw1_ref, z_ref):
    i = pl.program_id(0)
    accf = jnp.concatenate([acc_ref[0], acc_ref[1]], axis=1)
    yf = jnp.concatenate([y2_ref[0], y2_ref[1]], axis=1)
    dinv = _dinv_block(degp_ref[...], i)
    h = jnp.maximum(dinv * (accf + yf) + b0_ref[...], 0.0)
    hw1 = jnp.dot(h, w1_ref[...], preferred_element_type=jnp.float32)
    hw1_ref[...] = hw1
    z_ref[...] = hw1 * dinv


def _tc3_body(acc_ref, z_ref, degp_ref, b1_ref, out_ref):
    i = pl.program_id(0)
    accf = acc_ref[0] + acc_ref[1] + z_ref[...]
    dinv = _dinv_block(degp_ref[...], i)
    out_ref[...] = jnp.maximum(dinv * accf + b1_ref[...], 0.0)


_GRID = NP // BN


def _bs(shape, index_map):
    return pl.BlockSpec(shape, index_map)


_tc1 = pl.pallas_call(
    _tc1_body,
    grid=(_GRID,),
    in_specs=[
        _bs((BN, 128), lambda i: (i, 0)),
        _bs((BN, 2), lambda i: (i, 0)),
        _bs((128, 256), lambda i: (0, 0)),
    ],
    out_specs=_bs((2, BN, 128), lambda i: (0, i, 0)),
    out_shape=jax.ShapeDtypeStruct((2, NP, 128), jnp.float32),
)

_tc0 = pl.pallas_call(
    _tc0_body,
    grid=(_GRID,),
    in_specs=[
        _bs((BN, 128), lambda i: (i, 0)),
        _bs((128, 256), lambda i: (0, 0)),
        _bs((256, 128), lambda i: (0, 0)),
    ],
    out_specs=_bs((BN, 128), lambda i: (i, 0)),
    out_shape=jax.ShapeDtypeStruct((N, 128), jnp.float32),
)

_tcr = pl.pallas_call(
    _tcr_body,
    grid=(_GRID,),
    in_specs=[_bs((BN, 128), lambda i: (i, 0))],
    out_specs=_bs((BN, 128), lambda i: (i, 0)),
    out_shape=jax.ShapeDtypeStruct((N, 128), jnp.float32),
)

_tc2 = pl.pallas_call(
    _tc2_body,
    grid=(_GRID,),
    in_specs=[
        _bs((2, BN, 128), lambda i: (0, i, 0)),
        _bs((2, BN, 128), lambda i: (0, i, 0)),
        _bs((BN, 2), lambda i: (i, 0)),
        _bs((256, 128), lambda i: (0, 0)),
        _bs((1, 256), lambda i: (0, 0)),
    ],
    out_specs=[
        _bs((BN, 128), lambda i: (i, 0)),
        _bs((BN, 128), lambda i: (i, 0)),
    ],
    out_shape=[
        jax.ShapeDtypeStruct((NP, 128), jnp.float32),
        jax.ShapeDtypeStruct((NP, 128), jnp.float32),
    ],
)

_tc3 = pl.pallas_call(
    _tc3_body,
    grid=(_GRID,),
    in_specs=[
        _bs((2, BN, 128), lambda i: (0, i, 0)),
        _bs((BN, 128), lambda i: (i, 0)),
        _bs((BN, 2), lambda i: (i, 0)),
        _bs((1, 128), lambda i: (0, 0)),
    ],
    out_specs=_bs((BN, 128), lambda i: (i, 0)),
    out_shape=jax.ShapeDtypeStruct((N, 128), jnp.float32),
)


def kernel(x, edge_index, W0, b0, W1, b1):
    src = edge_index[0]
    dst = edge_index[1]

    pad = N + (jnp.arange(EPAD - E, dtype=jnp.int32) % (NP - N))
    srcp = jnp.concatenate([src, pad])
    dstp = jnp.concatenate([dst, pad])
    src2r = jnp.concatenate([srcp, srcp + NP]).reshape(2 * CH64, CHUNK)
    srcr = srcp.reshape(CH64, CHUNK)
    dstr = dstp.reshape(CH64, CHUNK)
    dstr128 = dstp.reshape(CH, 128)

    x_pad = jnp.concatenate(
        [x, jnp.zeros((NP - N, x.shape[1]), x.dtype)], axis=0)

    zeros_r = jnp.zeros((RPT,), jnp.float32)
    zeros_r128 = jnp.zeros((RPT, 128), jnp.float32)
    ones128 = jnp.ones((128,), jnp.float32)

    degf = _deg_kernel(dstr128, ones128, zeros_r)
    degp = jnp.stack([degf[:NP], degf[NP:]], axis=1)

    y2 = _tc1(x_pad, degp, W0)
    acc1 = _agg_feat(y2.reshape(2 * NP, 128), src2r, dstr, zeros_r128)
    x0 = _tc0(x_pad, W0, W1)
    hw1, z = _tc2(acc1.reshape(2, NP, 128), y2, degp, W1,
                  b0.reshape(1, 256))
    acc2 = _agg_edge(z, srcr, dstr, zeros_r128)
    h1 = _tcr(hw1)
    out = _tc3(acc2.reshape(2, NP, 128), z, degp, b1.reshape(1, 128))
    return (out, x0, h1)
